# baseline (speedup 1.0000x reference)
import functools

import jax
import jax.numpy as jnp
from jax import lax
from jax.experimental import pallas as pl
from jax.experimental.pallas import tpu as pltpu
from jax.experimental.pallas import tpu_sc as plsc

N_USER = 20000
N_ITEM = 30000
N_NODES = N_USER + N_ITEM
N_PAD = 50048
EMB = 64
HALF = 32
N_EDGES = 800000
N_TILES = 16
ROWS_PT = N_PAD // N_TILES
CHUNK = 128
NBUF = 4
CHUNKS_PER_BLOCK = 28
BLOCK_E = CHUNK * CHUNKS_PER_BLOCK
BLOCKS = 14
EDGES_PT = BLOCK_E * BLOCKS
E_ROWS = N_EDGES // CHUNK
TAIL_CHUNKS = (N_EDGES - (N_TILES - 1) * EDGES_PT
               - (BLOCKS - 1) * BLOCK_E) // CHUNK
WB_FULL = ROWS_PT // CHUNK
WB_TAIL = ROWS_PT - WB_FULL * CHUNK
U_CHUNKS = N_USER // CHUNK
U_TAIL = N_USER - U_CHUNKS * CHUNK
I_CHUNKS = N_ITEM // CHUNK
I_TAIL = N_ITEM - I_CHUNKS * CHUNK
LANES = 16


def _sc_body(user3, item3, er3, wr,
             user_o, item_o, sum_slab, ego_scr, spmem,
             src_blk, w_blk, dst_blk, gb0, gb1, gb2, gb3,
             gs0, gs1, gs2, gs3, ss0, ss1, ss2, ss3):
    c = lax.axis_index("c")
    s = lax.axis_index("s")
    wid = s * 2 + c
    col_off = c * N_PAD
    offv = jnp.broadcast_to(col_off, (LANES,)).astype(jnp.int32)
    third = jnp.float32(1.0 / 3.0)
    zv = jnp.zeros((LANES,), jnp.float32)
    gbufs = [gb0, gb1, gb2, gb3]
    gsems = [gs0, gs1, gs2, gs3]
    ssems = [ss0, ss1, ss2, ss3]
    ebuf, abuf = gb0, gb1

    def copy_rows(tbl, kid, slab_base, rows, bufp):
        r0 = kid * CHUNK
        for cc in range(2):
            pltpu.sync_copy(tbl.at[pl.ds(r0, rows), pl.ds(cc * HALF, HALF)],
                            bufp.at[pl.ds(0, rows)])
            pltpu.sync_copy(
                bufp.at[pl.ds(0, rows)],
                ego_scr.at[pl.ds(cc * N_PAD + slab_base + r0, rows)])

    for it in range(5):
        kid = it * 32 + wid

        @pl.when(kid < U_CHUNKS)
        def _():
            copy_rows(user3, kid, 0, CHUNK, gb0)

        @pl.when(kid == U_CHUNKS)
        def _():
            copy_rows(user3, kid, 0, U_TAIL, gb0)

    for it in range(8):
        kid = it * 32 + wid

        @pl.when(kid < I_CHUNKS)
        def _():
            copy_rows(item3, kid, N_USER, CHUNK, gb1)

        @pl.when(kid == I_CHUNKS)
        def _():
            copy_rows(item3, kid, N_USER, I_TAIL, gb1)

    def layer(first, last):
        def zfill(i, _):
            for h in range(2):
                ebuf[i, pl.ds(h * LANES, LANES)] = zv
            return 0

        lax.fori_loop(0, CHUNK, zfill, 0)

        def zcopy(k, _):
            pltpu.sync_copy(ebuf,
                            spmem.at[pl.ds(s * ROWS_PT + k * CHUNK, CHUNK)])
            return 0

        lax.fori_loop(0, WB_FULL, zcopy, 0)
        pltpu.sync_copy(
            ebuf.at[pl.ds(0, WB_TAIL)],
            spmem.at[pl.ds(s * ROWS_PT + WB_FULL * CHUNK, WB_TAIL)])
        plsc.subcore_barrier()

        def gidx(j):
            return src_blk.at[j]

        def stage_block(row0, rows, edges):
            d1 = pltpu.async_copy(er3.at[0, pl.ds(row0, rows)],
                                  src_blk.at[pl.ds(0, rows)], gs1)
            d2 = pltpu.async_copy(wr.at[pl.ds(row0 * CHUNK, edges)],
                                  w_blk.at[pl.ds(0, edges)], gs2)
            d3 = pltpu.async_copy(er3.at[1, pl.ds(row0, rows)],
                                  dst_blk.at[pl.ds(0, rows)], gs3)
            d1.wait()
            d2.wait()
            d3.wait()

            def add_off(jj, _):
                for i in range(CHUNK // LANES):
                    src_blk[jj, pl.ds(i * LANES, LANES)] = (
                        src_blk[jj, pl.ds(i * LANES, LANES)] + offv)
                return 0

            lax.fori_loop(0, rows, add_off, 0)

        def block_body(b, _):
            row0 = (s * EDGES_PT + b * BLOCK_E) // CHUNK
            stage_block(row0, CHUNKS_PER_BLOCK, BLOCK_E)

            for p in range(NBUF):
                pltpu.async_copy(ego_scr.at[gidx(p)], gbufs[p], gsems[p])

            def quad_body(jq, _):
                for p in range(NBUF):
                    j = NBUF * jq + p
                    pltpu.make_async_copy(
                        ego_scr.at[gidx(j)], gbufs[p], gsems[p]).wait()

                    def mul_grp(g, _, p=p, j=j):
                        wvec = w_blk[pl.ds(j * CHUNK + g * LANES, LANES)]
                        for i in range(LANES):
                            row = g * LANES + i
                            wv = jnp.broadcast_to(wvec[i], (LANES,))
                            for h in range(2):
                                gbufs[p][row, pl.ds(h * LANES, LANES)] = (
                                    gbufs[p][row, pl.ds(h * LANES, LANES)]
                                    * wv)
                        return 0

                    lax.fori_loop(0, CHUNK // LANES, mul_grp, 0)
                    pltpu.async_copy(gbufs[p], spmem.at[dst_blk.at[j]],
                                     ssems[p], add=True)

                    @pl.when(j + NBUF < CHUNKS_PER_BLOCK)
                    def _():
                        pltpu.make_async_copy(
                            gbufs[p], spmem.at[dst_blk.at[j]],
                            ssems[p]).wait()
                        pltpu.async_copy(ego_scr.at[gidx(j + NBUF)],
                                         gbufs[p], gsems[p])
                return 0

            lax.fori_loop(0, CHUNKS_PER_BLOCK // NBUF, quad_body, 0)
            for p in range(NBUF):
                j = CHUNKS_PER_BLOCK - NBUF + p
                pltpu.make_async_copy(
                    gbufs[p], spmem.at[dst_blk.at[j]], ssems[p]).wait()
            return 0

        nblocks = jnp.where(s == N_TILES - 1, BLOCKS - 1, BLOCKS)
        lax.fori_loop(0, nblocks, block_body, 0)

        @pl.when(s == N_TILES - 1)
        def _():
            stage_block(((N_TILES - 1) * EDGES_PT
                         + (BLOCKS - 1) * BLOCK_E) // CHUNK,
                        TAIL_CHUNKS, TAIL_CHUNKS * CHUNK)

            def tail_body(j, _):
                pltpu.async_copy(ego_scr.at[gidx(j)], gb0, gs0).wait()

                def mul_grp(g, _):
                    wvec = w_blk[pl.ds(j * CHUNK + g * LANES, LANES)]
                    for i in range(LANES):
                        row = g * LANES + i
                        wv = jnp.broadcast_to(wvec[i], (LANES,))
                        for h in range(2):
                            gb0[row, pl.ds(h * LANES, LANES)] = (
                                gb0[row, pl.ds(h * LANES, LANES)] * wv)
                    return 0

                lax.fori_loop(0, CHUNK // LANES, mul_grp, 0)
                pltpu.sync_copy(gb0, spmem.at[dst_blk.at[j]], add=True)
                return 0

            lax.fori_loop(0, TAIL_CHUNKS, tail_body, 0)

        plsc.subcore_barrier()

        if not last:
            def wb_chunk(r0, rows):
                pltpu.sync_copy(spmem.at[pl.ds(r0, rows)],
                                ebuf.at[pl.ds(0, rows)])
                pltpu.sync_copy(ebuf.at[pl.ds(0, rows)],
                                ego_scr.at[pl.ds(col_off + r0, rows)])
                if not first:
                    pltpu.sync_copy(sum_slab.at[pl.ds(col_off + r0, rows)],
                                    abuf.at[pl.ds(0, rows)])

                def wb_row(i, _):
                    for h in range(2):
                        v = ebuf[i, pl.ds(h * LANES, LANES)] * third
                        if first:
                            abuf[i, pl.ds(h * LANES, LANES)] = v
                        else:
                            abuf[i, pl.ds(h * LANES, LANES)] = (
                                abuf[i, pl.ds(h * LANES, LANES)] + v)
                    return 0

                lax.fori_loop(0, rows, wb_row, 0)
                pltpu.sync_copy(abuf.at[pl.ds(0, rows)],
                                sum_slab.at[pl.ds(col_off + r0, rows)])

            def wb_body(k, _):
                wb_chunk(s * ROWS_PT + k * CHUNK, CHUNK)
                return 0

            lax.fori_loop(0, WB_FULL, wb_body, 0)
            wb_chunk(s * ROWS_PT + WB_FULL * CHUNK, WB_TAIL)
        else:
            def fin_chunk(out_ref, kid, node_base, rows):
                r0 = node_base + kid * CHUNK
                pltpu.sync_copy(spmem.at[pl.ds(r0, rows)],
                                ebuf.at[pl.ds(0, rows)])
                pltpu.sync_copy(sum_slab.at[pl.ds(col_off + r0, rows)],
                                abuf.at[pl.ds(0, rows)])

                def fin_row(i, _):
                    for h in range(2):
                        abuf[i, pl.ds(h * LANES, LANES)] = (
                            abuf[i, pl.ds(h * LANES, LANES)]
                            + ebuf[i, pl.ds(h * LANES, LANES)] * third)
                    return 0

                lax.fori_loop(0, rows, fin_row, 0)
                pltpu.sync_copy(
                    abuf.at[pl.ds(0, rows)],
                    out_ref.at[pl.ds(kid * CHUNK, rows),
                               pl.ds(c * HALF, HALF)])

            def ubody(it, _):
                kid = it * N_TILES + s

                @pl.when(kid < U_CHUNKS)
                def _():
                    fin_chunk(user_o, kid, 0, CHUNK)

                @pl.when(kid == U_CHUNKS)
                def _():
                    fin_chunk(user_o, kid, 0, U_TAIL)
                return 0

            lax.fori_loop(0, (U_CHUNKS + N_TILES) // N_TILES, ubody, 0)

            def ibody(it, _):
                kid = it * N_TILES + s

                @pl.when(kid < I_CHUNKS)
                def _():
                    fin_chunk(item_o, kid, N_USER, CHUNK)

                @pl.when(kid == I_CHUNKS)
                def _():
                    fin_chunk(item_o, kid, N_USER, I_TAIL)
                return 0

            lax.fori_loop(0, (I_CHUNKS + N_TILES) // N_TILES, ibody, 0)

    layer(first=True, last=False)
    plsc.subcore_barrier()
    layer(first=False, last=False)
    plsc.subcore_barrier()
    layer(first=False, last=True)


_sc_kernel = functools.partial(
    pl.kernel,
    out_type=(
        jax.ShapeDtypeStruct((N_USER, EMB), jnp.float32),
        jax.ShapeDtypeStruct((N_ITEM, EMB), jnp.float32),
        jax.ShapeDtypeStruct((2 * N_PAD, HALF), jnp.float32),
        jax.ShapeDtypeStruct((2 * N_PAD, HALF), jnp.float32),
    ),
    mesh=plsc.VectorSubcoreMesh(core_axis_name="c", subcore_axis_name="s"),
    compiler_params=pltpu.CompilerParams(use_tc_tiling_on_sc=False),
    scratch_types=[
        pltpu.VMEM_SHARED((N_PAD, HALF), jnp.float32),
        pltpu.VMEM((CHUNKS_PER_BLOCK, CHUNK), jnp.int32),
        pltpu.VMEM((BLOCK_E,), jnp.float32),
        pltpu.VMEM((CHUNKS_PER_BLOCK, CHUNK), jnp.int32),
        pltpu.VMEM((CHUNK, HALF), jnp.float32),
        pltpu.VMEM((CHUNK, HALF), jnp.float32),
        pltpu.VMEM((CHUNK, HALF), jnp.float32),
        pltpu.VMEM((CHUNK, HALF), jnp.float32),
        pltpu.SemaphoreType.DMA,
        pltpu.SemaphoreType.DMA,
        pltpu.SemaphoreType.DMA,
        pltpu.SemaphoreType.DMA,
        pltpu.SemaphoreType.DMA,
        pltpu.SemaphoreType.DMA,
        pltpu.SemaphoreType.DMA,
        pltpu.SemaphoreType.DMA,
    ],
)(_sc_body)


def kernel(user_emb, item_emb, edge_index, edge_weight):
    er3 = edge_index.reshape(2, E_ROWS, CHUNK)
    user_o, item_o, _, _ = _sc_kernel(user_emb, item_emb, er3, edge_weight)
    return user_o, item_o


# speedup vs baseline: 1.0425x; 1.0425x over previous
import functools

import jax
import jax.numpy as jnp
from jax import lax
from jax.experimental import pallas as pl
from jax.experimental.pallas import tpu as pltpu
from jax.experimental.pallas import tpu_sc as plsc

N_USER = 20000
N_ITEM = 30000
N_NODES = N_USER + N_ITEM
N_PAD = 50048
EMB = 64
HALF = 32
N_EDGES = 800000
N_TILES = 16
ROWS_PT = N_PAD // N_TILES
CHUNK = 128
NBUF = 4
CHUNKS_PER_BLOCK = 28
BLOCK_E = CHUNK * CHUNKS_PER_BLOCK
BLOCKS = 14
EDGES_PT = BLOCK_E * BLOCKS
E_ROWS = N_EDGES // CHUNK
TAIL_CHUNKS = (N_EDGES - (N_TILES - 1) * EDGES_PT
               - (BLOCKS - 1) * BLOCK_E) // CHUNK
WB_FULL = ROWS_PT // CHUNK
WB_TAIL = ROWS_PT - WB_FULL * CHUNK
U_CHUNKS = N_USER // CHUNK
U_TAIL = N_USER - U_CHUNKS * CHUNK
I_CHUNKS = N_ITEM // CHUNK
I_TAIL = N_ITEM - I_CHUNKS * CHUNK
LANES = 16


def _sc_body(user3, item3, er3, wr,
             user_o, item_o, sum_slab, ego_scr, spmem,
             src_blk, w_blk, dst_blk, gb0, gb1, gb2, gb3,
             gs0, gs1, gs2, gs3, ss0, ss1, ss2, ss3):
    c = lax.axis_index("c")
    s = lax.axis_index("s")
    wid = s * 2 + c
    col_off = c * N_PAD
    offv = jnp.broadcast_to(col_off, (LANES,)).astype(jnp.int32)
    third = jnp.float32(1.0 / 3.0)
    zv = jnp.zeros((LANES,), jnp.float32)
    gbufs = [gb0, gb1, gb2, gb3]
    gsems = [gs0, gs1, gs2, gs3]
    ssems = [ss0, ss1, ss2, ss3]
    ebuf, abuf = gb0, gb1

    def copy_rows(tbl, kid, slab_base, rows, bufp):
        r0 = kid * CHUNK
        for cc in range(2):
            pltpu.sync_copy(tbl.at[pl.ds(r0, rows), pl.ds(cc * HALF, HALF)],
                            bufp.at[pl.ds(0, rows)])
            pltpu.sync_copy(
                bufp.at[pl.ds(0, rows)],
                ego_scr.at[pl.ds(cc * N_PAD + slab_base + r0, rows)])

    for it in range(5):
        kid = it * 32 + wid

        @pl.when(kid < U_CHUNKS)
        def _():
            copy_rows(user3, kid, 0, CHUNK, gb0)

        @pl.when(kid == U_CHUNKS)
        def _():
            copy_rows(user3, kid, 0, U_TAIL, gb0)

    for it in range(8):
        kid = it * 32 + wid

        @pl.when(kid < I_CHUNKS)
        def _():
            copy_rows(item3, kid, N_USER, CHUNK, gb1)

        @pl.when(kid == I_CHUNKS)
        def _():
            copy_rows(item3, kid, N_USER, I_TAIL, gb1)

    def layer(first, last):
        def zfill(i, _):
            for h in range(2):
                ebuf[i, pl.ds(h * LANES, LANES)] = zv
            return 0

        lax.fori_loop(0, CHUNK, zfill, 0)

        def zcopy(k, _):
            pltpu.sync_copy(ebuf,
                            spmem.at[pl.ds(s * ROWS_PT + k * CHUNK, CHUNK)])
            return 0

        lax.fori_loop(0, WB_FULL, zcopy, 0)
        pltpu.sync_copy(
            ebuf.at[pl.ds(0, WB_TAIL)],
            spmem.at[pl.ds(s * ROWS_PT + WB_FULL * CHUNK, WB_TAIL)])
        plsc.subcore_barrier()

        def gidx(j):
            return src_blk.at[j]

        def stage_block(row0, rows, edges):
            d1 = pltpu.async_copy(er3.at[0, pl.ds(row0, rows)],
                                  src_blk.at[pl.ds(0, rows)], gs1)
            d2 = pltpu.async_copy(wr.at[pl.ds(row0 * CHUNK, edges)],
                                  w_blk.at[pl.ds(0, edges)], gs2)
            d3 = pltpu.async_copy(er3.at[1, pl.ds(row0, rows)],
                                  dst_blk.at[pl.ds(0, rows)], gs3)
            d1.wait()
            d2.wait()
            d3.wait()

            def add_off(jj, _):
                for i in range(CHUNK // LANES):
                    src_blk[jj, pl.ds(i * LANES, LANES)] = (
                        src_blk[jj, pl.ds(i * LANES, LANES)] + offv)
                return 0

            lax.fori_loop(0, rows, add_off, 0)

        def block_body(b, _):
            row0 = (s * EDGES_PT + b * BLOCK_E) // CHUNK
            stage_block(row0, CHUNKS_PER_BLOCK, BLOCK_E)

            for p in range(NBUF):
                pltpu.async_copy(ego_scr.at[gidx(p)], gbufs[p], gsems[p])

            def quad_body(jq, _):
                for p in range(NBUF):
                    j = NBUF * jq + p
                    pltpu.make_async_copy(
                        ego_scr.at[gidx(j)], gbufs[p], gsems[p]).wait()

                    def mul_grp(g, _, p=p, j=j):
                        wvec = w_blk[pl.ds(j * CHUNK + g * LANES, LANES)]
                        for i in range(LANES):
                            row = g * LANES + i
                            wv = jnp.broadcast_to(wvec[i], (LANES,))
                            for h in range(2):
                                gbufs[p][row, pl.ds(h * LANES, LANES)] = (
                                    gbufs[p][row, pl.ds(h * LANES, LANES)]
                                    * wv)
                        return 0

                    lax.fori_loop(0, CHUNK // LANES, mul_grp, 0)
                    pltpu.async_copy(gbufs[p], spmem.at[dst_blk.at[j]],
                                     ssems[p], add=True)

                    @pl.when(j + NBUF < CHUNKS_PER_BLOCK)
                    def _():
                        pltpu.make_async_copy(
                            gbufs[p], spmem.at[dst_blk.at[j]],
                            ssems[p]).wait()
                        pltpu.async_copy(ego_scr.at[gidx(j + NBUF)],
                                         gbufs[p], gsems[p])
                return 0

            lax.fori_loop(0, CHUNKS_PER_BLOCK // NBUF, quad_body, 0)
            for p in range(NBUF):
                j = CHUNKS_PER_BLOCK - NBUF + p
                pltpu.make_async_copy(
                    gbufs[p], spmem.at[dst_blk.at[j]], ssems[p]).wait()
            return 0

        nblocks = jnp.where(s == N_TILES - 1, BLOCKS - 1, BLOCKS)
        lax.fori_loop(0, nblocks, block_body, 0)

        @pl.when(s == N_TILES - 1)
        def _():
            stage_block(((N_TILES - 1) * EDGES_PT
                         + (BLOCKS - 1) * BLOCK_E) // CHUNK,
                        TAIL_CHUNKS, TAIL_CHUNKS * CHUNK)

            def tail_body(j, _):
                pltpu.async_copy(ego_scr.at[gidx(j)], gb0, gs0).wait()

                def mul_grp(g, _):
                    wvec = w_blk[pl.ds(j * CHUNK + g * LANES, LANES)]
                    for i in range(LANES):
                        row = g * LANES + i
                        wv = jnp.broadcast_to(wvec[i], (LANES,))
                        for h in range(2):
                            gb0[row, pl.ds(h * LANES, LANES)] = (
                                gb0[row, pl.ds(h * LANES, LANES)] * wv)
                    return 0

                lax.fori_loop(0, CHUNK // LANES, mul_grp, 0)
                pltpu.sync_copy(gb0, spmem.at[dst_blk.at[j]], add=True)
                return 0

            lax.fori_loop(0, TAIL_CHUNKS, tail_body, 0)

        plsc.subcore_barrier()

        if not last:
            def wb_r0(k):
                return s * ROWS_PT + k * CHUNK

            def wb_issue_loads(k, q):
                pltpu.async_copy(spmem.at[pl.ds(wb_r0(k), CHUNK)],
                                 gbufs[2 * q], gsems[2 * q])
                if not first:
                    pltpu.async_copy(
                        sum_slab.at[pl.ds(col_off + wb_r0(k), CHUNK)],
                        gbufs[2 * q + 1], gsems[2 * q + 1])

            for q in range(2):
                wb_issue_loads(q, q)

            def wb_pair(k2, _):
                for q in range(2):
                    k = 2 * k2 + q
                    eb, ab = gbufs[2 * q], gbufs[2 * q + 1]
                    pltpu.make_async_copy(
                        spmem.at[pl.ds(wb_r0(k), CHUNK)], eb,
                        gsems[2 * q]).wait()
                    if not first:
                        pltpu.make_async_copy(
                            sum_slab.at[pl.ds(col_off + wb_r0(k), CHUNK)],
                            ab, gsems[2 * q + 1]).wait()

                    def wb_row(i, _, eb=eb, ab=ab):
                        for h in range(2):
                            v = eb[i, pl.ds(h * LANES, LANES)] * third
                            if first:
                                ab[i, pl.ds(h * LANES, LANES)] = v
                            else:
                                ab[i, pl.ds(h * LANES, LANES)] = (
                                    ab[i, pl.ds(h * LANES, LANES)] + v)
                        return 0

                    lax.fori_loop(0, CHUNK, wb_row, 0)
                    pltpu.async_copy(eb,
                                     ego_scr.at[pl.ds(col_off + wb_r0(k),
                                                      CHUNK)],
                                     ssems[2 * q])
                    pltpu.async_copy(ab,
                                     sum_slab.at[pl.ds(col_off + wb_r0(k),
                                                       CHUNK)],
                                     ssems[2 * q + 1])

                    @pl.when(k + 2 < WB_FULL)
                    def _(k=k, q=q, eb=eb, ab=ab):
                        pltpu.make_async_copy(
                            eb, ego_scr.at[pl.ds(col_off + wb_r0(k), CHUNK)],
                            ssems[2 * q]).wait()
                        pltpu.make_async_copy(
                            ab, sum_slab.at[pl.ds(col_off + wb_r0(k), CHUNK)],
                            ssems[2 * q + 1]).wait()
                        wb_issue_loads(k + 2, q)
                return 0

            lax.fori_loop(0, WB_FULL // 2, wb_pair, 0)
            for q in range(2):
                k = WB_FULL - 2 + q
                pltpu.make_async_copy(
                    gbufs[2 * q],
                    ego_scr.at[pl.ds(col_off + wb_r0(k), CHUNK)],
                    ssems[2 * q]).wait()
                pltpu.make_async_copy(
                    gbufs[2 * q + 1],
                    sum_slab.at[pl.ds(col_off + wb_r0(k), CHUNK)],
                    ssems[2 * q + 1]).wait()
            r0 = s * ROWS_PT + WB_FULL * CHUNK
            pltpu.sync_copy(spmem.at[pl.ds(r0, WB_TAIL)],
                            ebuf.at[pl.ds(0, WB_TAIL)])
            if not first:
                pltpu.sync_copy(sum_slab.at[pl.ds(col_off + r0, WB_TAIL)],
                                abuf.at[pl.ds(0, WB_TAIL)])

            def wb_row_t(i, _):
                for h in range(2):
                    v = ebuf[i, pl.ds(h * LANES, LANES)] * third
                    if first:
                        abuf[i, pl.ds(h * LANES, LANES)] = v
                    else:
                        abuf[i, pl.ds(h * LANES, LANES)] = (
                            abuf[i, pl.ds(h * LANES, LANES)] + v)
                return 0

            lax.fori_loop(0, WB_TAIL, wb_row_t, 0)
            pltpu.sync_copy(ebuf.at[pl.ds(0, WB_TAIL)],
                            ego_scr.at[pl.ds(col_off + r0, WB_TAIL)])
            pltpu.sync_copy(abuf.at[pl.ds(0, WB_TAIL)],
                            sum_slab.at[pl.ds(col_off + r0, WB_TAIL)])
        else:
            def fin_chunk(out_ref, kid, node_base, rows):
                r0 = node_base + kid * CHUNK
                pltpu.sync_copy(spmem.at[pl.ds(r0, rows)],
                                ebuf.at[pl.ds(0, rows)])
                pltpu.sync_copy(sum_slab.at[pl.ds(col_off + r0, rows)],
                                abuf.at[pl.ds(0, rows)])

                def fin_row(i, _):
                    for h in range(2):
                        abuf[i, pl.ds(h * LANES, LANES)] = (
                            abuf[i, pl.ds(h * LANES, LANES)]
                            + ebuf[i, pl.ds(h * LANES, LANES)] * third)
                    return 0

                lax.fori_loop(0, rows, fin_row, 0)
                pltpu.sync_copy(
                    abuf.at[pl.ds(0, rows)],
                    out_ref.at[pl.ds(kid * CHUNK, rows),
                               pl.ds(c * HALF, HALF)])

            def ubody(it, _):
                kid = it * N_TILES + s

                @pl.when(kid < U_CHUNKS)
                def _():
                    fin_chunk(user_o, kid, 0, CHUNK)

                @pl.when(kid == U_CHUNKS)
                def _():
                    fin_chunk(user_o, kid, 0, U_TAIL)
                return 0

            lax.fori_loop(0, (U_CHUNKS + N_TILES) // N_TILES, ubody, 0)

            def ibody(it, _):
                kid = it * N_TILES + s

                @pl.when(kid < I_CHUNKS)
                def _():
                    fin_chunk(item_o, kid, N_USER, CHUNK)

                @pl.when(kid == I_CHUNKS)
                def _():
                    fin_chunk(item_o, kid, N_USER, I_TAIL)
                return 0

            lax.fori_loop(0, (I_CHUNKS + N_TILES) // N_TILES, ibody, 0)

    layer(first=True, last=False)
    plsc.subcore_barrier()
    layer(first=False, last=False)
    plsc.subcore_barrier()
    layer(first=False, last=True)


_sc_kernel = functools.partial(
    pl.kernel,
    out_type=(
        jax.ShapeDtypeStruct((N_USER, EMB), jnp.float32),
        jax.ShapeDtypeStruct((N_ITEM, EMB), jnp.float32),
        jax.ShapeDtypeStruct((2 * N_PAD, HALF), jnp.float32),
        jax.ShapeDtypeStruct((2 * N_PAD, HALF), jnp.float32),
    ),
    mesh=plsc.VectorSubcoreMesh(core_axis_name="c", subcore_axis_name="s"),
    compiler_params=pltpu.CompilerParams(use_tc_tiling_on_sc=False),
    scratch_types=[
        pltpu.VMEM_SHARED((N_PAD, HALF), jnp.float32),
        pltpu.VMEM((CHUNKS_PER_BLOCK, CHUNK), jnp.int32),
        pltpu.VMEM((BLOCK_E,), jnp.float32),
        pltpu.VMEM((CHUNKS_PER_BLOCK, CHUNK), jnp.int32),
        pltpu.VMEM((CHUNK, HALF), jnp.float32),
        pltpu.VMEM((CHUNK, HALF), jnp.float32),
        pltpu.VMEM((CHUNK, HALF), jnp.float32),
        pltpu.VMEM((CHUNK, HALF), jnp.float32),
        pltpu.SemaphoreType.DMA,
        pltpu.SemaphoreType.DMA,
        pltpu.SemaphoreType.DMA,
        pltpu.SemaphoreType.DMA,
        pltpu.SemaphoreType.DMA,
        pltpu.SemaphoreType.DMA,
        pltpu.SemaphoreType.DMA,
        pltpu.SemaphoreType.DMA,
    ],
)(_sc_body)


def kernel(user_emb, item_emb, edge_index, edge_weight):
    er3 = edge_index.reshape(2, E_ROWS, CHUNK)
    user_o, item_o, _, _ = _sc_kernel(user_emb, item_emb, er3, edge_weight)
    return user_o, item_o


# speedup vs baseline: 1.1171x; 1.0716x over previous
import functools

import jax
import jax.numpy as jnp
from jax import lax
from jax.experimental import pallas as pl
from jax.experimental.pallas import tpu as pltpu
from jax.experimental.pallas import tpu_sc as plsc

N_USER = 20000
N_ITEM = 30000
N_NODES = N_USER + N_ITEM
N_PAD = 50048
EMB = 64
HALF = 32
N_EDGES = 800000
N_TILES = 16
ROWS_PT = N_PAD // N_TILES
CHUNK = 128
NBUF = 4
CHUNKS_PER_BLOCK = 28
BLOCK_E = CHUNK * CHUNKS_PER_BLOCK
BLOCKS = 14
EDGES_PT = BLOCK_E * BLOCKS
E_ROWS = N_EDGES // CHUNK
TAIL_CHUNKS = (N_EDGES - (N_TILES - 1) * EDGES_PT
               - (BLOCKS - 1) * BLOCK_E) // CHUNK
WB_FULL = ROWS_PT // CHUNK
WB_TAIL = ROWS_PT - WB_FULL * CHUNK
U_CHUNKS = N_USER // CHUNK
U_TAIL = N_USER - U_CHUNKS * CHUNK
I_CHUNKS = N_ITEM // CHUNK
I_TAIL = N_ITEM - I_CHUNKS * CHUNK
LANES = 16


def _sc_body(user3, item3, er3, wr,
             user_o, item_o, sum_slab, ego_scr, spmem,
             src_blk, w_blk, dst_blk, gb0, gb1, gb2, gb3,
             gs0, gs1, gs2, gs3, ss0, ss1, ss2, ss3):
    c = lax.axis_index("c")
    s = lax.axis_index("s")
    wid = s * 2 + c
    col_off = c * N_PAD
    offv = jnp.broadcast_to(col_off, (LANES,)).astype(jnp.int32)
    third = jnp.float32(1.0 / 3.0)
    zv = jnp.zeros((LANES,), jnp.float32)
    gbufs = [gb0, gb1, gb2, gb3]
    gsems = [gs0, gs1, gs2, gs3]
    ssems = [ss0, ss1, ss2, ss3]
    ebuf, abuf = gb0, gb1

    def copy_rows(tbl, kid, slab_base, rows, bufp):
        r0 = kid * CHUNK
        for cc in range(2):
            pltpu.sync_copy(tbl.at[pl.ds(r0, rows), pl.ds(cc * HALF, HALF)],
                            bufp.at[pl.ds(0, rows)])
            pltpu.sync_copy(
                bufp.at[pl.ds(0, rows)],
                ego_scr.at[pl.ds(cc * N_PAD + slab_base + r0, rows)])

    for it in range(5):
        kid = it * 32 + wid

        @pl.when(kid < U_CHUNKS)
        def _():
            copy_rows(user3, kid, 0, CHUNK, gb0)

        @pl.when(kid == U_CHUNKS)
        def _():
            copy_rows(user3, kid, 0, U_TAIL, gb0)

    for it in range(8):
        kid = it * 32 + wid

        @pl.when(kid < I_CHUNKS)
        def _():
            copy_rows(item3, kid, N_USER, CHUNK, gb1)

        @pl.when(kid == I_CHUNKS)
        def _():
            copy_rows(item3, kid, N_USER, I_TAIL, gb1)

    def layer(first, last):
        def zfill(i, _):
            for h in range(2):
                ebuf[i, pl.ds(h * LANES, LANES)] = zv
            return 0

        lax.fori_loop(0, CHUNK, zfill, 0)

        def zcopy(k4, _):
            for p in range(4):
                pltpu.async_copy(
                    ebuf,
                    spmem.at[pl.ds(s * ROWS_PT + (4 * k4 + p) * CHUNK,
                                   CHUNK)],
                    gsems[p])
            return 0

        lax.fori_loop(0, WB_FULL // 4, zcopy, 0)
        pltpu.sync_copy(
            ebuf.at[pl.ds(0, WB_TAIL)],
            spmem.at[pl.ds(s * ROWS_PT + WB_FULL * CHUNK, WB_TAIL)])

        def zdrain(k4, _):
            for p in range(4):
                pltpu.make_async_copy(
                    ebuf,
                    spmem.at[pl.ds(s * ROWS_PT + (4 * k4 + p) * CHUNK,
                                   CHUNK)],
                    gsems[p]).wait()
            return 0

        lax.fori_loop(0, WB_FULL // 4, zdrain, 0)
        plsc.subcore_barrier()

        def gidx(j):
            return src_blk.at[j]

        def stage_block(row0, rows, edges):
            d1 = pltpu.async_copy(er3.at[0, pl.ds(row0, rows)],
                                  src_blk.at[pl.ds(0, rows)], gs1)
            d2 = pltpu.async_copy(wr.at[pl.ds(row0 * CHUNK, edges)],
                                  w_blk.at[pl.ds(0, edges)], gs2)
            d3 = pltpu.async_copy(er3.at[1, pl.ds(row0, rows)],
                                  dst_blk.at[pl.ds(0, rows)], gs3)
            d1.wait()
            d2.wait()
            d3.wait()

            def add_off(jj, _):
                for i in range(CHUNK // LANES):
                    src_blk[jj, pl.ds(i * LANES, LANES)] = (
                        src_blk[jj, pl.ds(i * LANES, LANES)] + offv)
                return 0

            lax.fori_loop(0, rows, add_off, 0)

        def block_body(b, _):
            row0 = (s * EDGES_PT + b * BLOCK_E) // CHUNK
            stage_block(row0, CHUNKS_PER_BLOCK, BLOCK_E)

            for p in range(NBUF):
                pltpu.async_copy(ego_scr.at[gidx(p)], gbufs[p], gsems[p])

            def quad_body(jq, _):
                for p in range(NBUF):
                    j = NBUF * jq + p
                    pltpu.make_async_copy(
                        ego_scr.at[gidx(j)], gbufs[p], gsems[p]).wait()

                    def mul_grp(g, _, p=p, j=j):
                        wvec = w_blk[pl.ds(j * CHUNK + g * LANES, LANES)]
                        for i in range(LANES):
                            row = g * LANES + i
                            wv = jnp.broadcast_to(wvec[i], (LANES,))
                            for h in range(2):
                                gbufs[p][row, pl.ds(h * LANES, LANES)] = (
                                    gbufs[p][row, pl.ds(h * LANES, LANES)]
                                    * wv)
                        return 0

                    lax.fori_loop(0, CHUNK // LANES, mul_grp, 0)
                    pltpu.async_copy(gbufs[p], spmem.at[dst_blk.at[j]],
                                     ssems[p], add=True)

                    @pl.when(j + NBUF < CHUNKS_PER_BLOCK)
                    def _():
                        pltpu.make_async_copy(
                            gbufs[p], spmem.at[dst_blk.at[j]],
                            ssems[p]).wait()
                        pltpu.async_copy(ego_scr.at[gidx(j + NBUF)],
                                         gbufs[p], gsems[p])
                return 0

            lax.fori_loop(0, CHUNKS_PER_BLOCK // NBUF, quad_body, 0)
            for p in range(NBUF):
                j = CHUNKS_PER_BLOCK - NBUF + p
                pltpu.make_async_copy(
                    gbufs[p], spmem.at[dst_blk.at[j]], ssems[p]).wait()
            return 0

        nblocks = jnp.where(s == N_TILES - 1, BLOCKS - 1, BLOCKS)
        lax.fori_loop(0, nblocks, block_body, 0)

        @pl.when(s == N_TILES - 1)
        def _():
            stage_block(((N_TILES - 1) * EDGES_PT
                         + (BLOCKS - 1) * BLOCK_E) // CHUNK,
                        TAIL_CHUNKS, TAIL_CHUNKS * CHUNK)

            def tail_body(j, _):
                pltpu.async_copy(ego_scr.at[gidx(j)], gb0, gs0).wait()

                def mul_grp(g, _):
                    wvec = w_blk[pl.ds(j * CHUNK + g * LANES, LANES)]
                    for i in range(LANES):
                        row = g * LANES + i
                        wv = jnp.broadcast_to(wvec[i], (LANES,))
                        for h in range(2):
                            gb0[row, pl.ds(h * LANES, LANES)] = (
                                gb0[row, pl.ds(h * LANES, LANES)] * wv)
                    return 0

                lax.fori_loop(0, CHUNK // LANES, mul_grp, 0)
                pltpu.sync_copy(gb0, spmem.at[dst_blk.at[j]], add=True)
                return 0

            lax.fori_loop(0, TAIL_CHUNKS, tail_body, 0)

        plsc.subcore_barrier()

        if not last:
            def wb_r0(k):
                return s * ROWS_PT + k * CHUNK

            def wb_issue_loads(k, q):
                pltpu.async_copy(spmem.at[pl.ds(wb_r0(k), CHUNK)],
                                 gbufs[2 * q], gsems[2 * q])
                if not first:
                    pltpu.async_copy(
                        sum_slab.at[pl.ds(col_off + wb_r0(k), CHUNK)],
                        gbufs[2 * q + 1], gsems[2 * q + 1])

            for q in range(2):
                wb_issue_loads(q, q)

            def wb_pair(k2, _):
                for q in range(2):
                    k = 2 * k2 + q
                    eb, ab = gbufs[2 * q], gbufs[2 * q + 1]
                    pltpu.make_async_copy(
                        spmem.at[pl.ds(wb_r0(k), CHUNK)], eb,
                        gsems[2 * q]).wait()
                    if not first:
                        pltpu.make_async_copy(
                            sum_slab.at[pl.ds(col_off + wb_r0(k), CHUNK)],
                            ab, gsems[2 * q + 1]).wait()

                    def wb_row(i, _, eb=eb, ab=ab):
                        for h in range(2):
                            v = eb[i, pl.ds(h * LANES, LANES)] * third
                            if first:
                                ab[i, pl.ds(h * LANES, LANES)] = v
                            else:
                                ab[i, pl.ds(h * LANES, LANES)] = (
                                    ab[i, pl.ds(h * LANES, LANES)] + v)
                        return 0

                    lax.fori_loop(0, CHUNK, wb_row, 0)
                    pltpu.async_copy(eb,
                                     ego_scr.at[pl.ds(col_off + wb_r0(k),
                                                      CHUNK)],
                                     ssems[2 * q])
                    pltpu.async_copy(ab,
                                     sum_slab.at[pl.ds(col_off + wb_r0(k),
                                                       CHUNK)],
                                     ssems[2 * q + 1])

                    @pl.when(k + 2 < WB_FULL)
                    def _(k=k, q=q, eb=eb, ab=ab):
                        pltpu.make_async_copy(
                            eb, ego_scr.at[pl.ds(col_off + wb_r0(k), CHUNK)],
                            ssems[2 * q]).wait()
                        pltpu.make_async_copy(
                            ab, sum_slab.at[pl.ds(col_off + wb_r0(k), CHUNK)],
                            ssems[2 * q + 1]).wait()
                        wb_issue_loads(k + 2, q)
                return 0

            lax.fori_loop(0, WB_FULL // 2, wb_pair, 0)
            for q in range(2):
                k = WB_FULL - 2 + q
                pltpu.make_async_copy(
                    gbufs[2 * q],
                    ego_scr.at[pl.ds(col_off + wb_r0(k), CHUNK)],
                    ssems[2 * q]).wait()
                pltpu.make_async_copy(
                    gbufs[2 * q + 1],
                    sum_slab.at[pl.ds(col_off + wb_r0(k), CHUNK)],
                    ssems[2 * q + 1]).wait()
            r0 = s * ROWS_PT + WB_FULL * CHUNK
            pltpu.sync_copy(spmem.at[pl.ds(r0, WB_TAIL)],
                            ebuf.at[pl.ds(0, WB_TAIL)])
            if not first:
                pltpu.sync_copy(sum_slab.at[pl.ds(col_off + r0, WB_TAIL)],
                                abuf.at[pl.ds(0, WB_TAIL)])

            def wb_row_t(i, _):
                for h in range(2):
                    v = ebuf[i, pl.ds(h * LANES, LANES)] * third
                    if first:
                        abuf[i, pl.ds(h * LANES, LANES)] = v
                    else:
                        abuf[i, pl.ds(h * LANES, LANES)] = (
                            abuf[i, pl.ds(h * LANES, LANES)] + v)
                return 0

            lax.fori_loop(0, WB_TAIL, wb_row_t, 0)
            pltpu.sync_copy(ebuf.at[pl.ds(0, WB_TAIL)],
                            ego_scr.at[pl.ds(col_off + r0, WB_TAIL)])
            pltpu.sync_copy(abuf.at[pl.ds(0, WB_TAIL)],
                            sum_slab.at[pl.ds(col_off + r0, WB_TAIL)])
        else:
            def fin_chunk(out_ref, kid, node_base, rows):
                r0 = node_base + kid * CHUNK
                pltpu.sync_copy(spmem.at[pl.ds(r0, rows)],
                                ebuf.at[pl.ds(0, rows)])
                pltpu.sync_copy(sum_slab.at[pl.ds(col_off + r0, rows)],
                                abuf.at[pl.ds(0, rows)])

                def fin_row(i, _):
                    for h in range(2):
                        abuf[i, pl.ds(h * LANES, LANES)] = (
                            abuf[i, pl.ds(h * LANES, LANES)]
                            + ebuf[i, pl.ds(h * LANES, LANES)] * third)
                    return 0

                lax.fori_loop(0, rows, fin_row, 0)
                pltpu.sync_copy(
                    abuf.at[pl.ds(0, rows)],
                    out_ref.at[pl.ds(kid * CHUNK, rows),
                               pl.ds(c * HALF, HALF)])

            def ubody(it, _):
                kid = it * N_TILES + s

                @pl.when(kid < U_CHUNKS)
                def _():
                    fin_chunk(user_o, kid, 0, CHUNK)

                @pl.when(kid == U_CHUNKS)
                def _():
                    fin_chunk(user_o, kid, 0, U_TAIL)
                return 0

            lax.fori_loop(0, (U_CHUNKS + N_TILES) // N_TILES, ubody, 0)

            def ibody(it, _):
                kid = it * N_TILES + s

                @pl.when(kid < I_CHUNKS)
                def _():
                    fin_chunk(item_o, kid, N_USER, CHUNK)

                @pl.when(kid == I_CHUNKS)
                def _():
                    fin_chunk(item_o, kid, N_USER, I_TAIL)
                return 0

            lax.fori_loop(0, (I_CHUNKS + N_TILES) // N_TILES, ibody, 0)

    layer(first=True, last=False)
    plsc.subcore_barrier()
    layer(first=False, last=False)
    plsc.subcore_barrier()
    layer(first=False, last=True)


_sc_kernel = functools.partial(
    pl.kernel,
    out_type=(
        jax.ShapeDtypeStruct((N_USER, EMB), jnp.float32),
        jax.ShapeDtypeStruct((N_ITEM, EMB), jnp.float32),
        jax.ShapeDtypeStruct((2 * N_PAD, HALF), jnp.float32),
        jax.ShapeDtypeStruct((2 * N_PAD, HALF), jnp.float32),
    ),
    mesh=plsc.VectorSubcoreMesh(core_axis_name="c", subcore_axis_name="s"),
    compiler_params=pltpu.CompilerParams(use_tc_tiling_on_sc=False),
    scratch_types=[
        pltpu.VMEM_SHARED((N_PAD, HALF), jnp.float32),
        pltpu.VMEM((CHUNKS_PER_BLOCK, CHUNK), jnp.int32),
        pltpu.VMEM((BLOCK_E,), jnp.float32),
        pltpu.VMEM((CHUNKS_PER_BLOCK, CHUNK), jnp.int32),
        pltpu.VMEM((CHUNK, HALF), jnp.float32),
        pltpu.VMEM((CHUNK, HALF), jnp.float32),
        pltpu.VMEM((CHUNK, HALF), jnp.float32),
        pltpu.VMEM((CHUNK, HALF), jnp.float32),
        pltpu.SemaphoreType.DMA,
        pltpu.SemaphoreType.DMA,
        pltpu.SemaphoreType.DMA,
        pltpu.SemaphoreType.DMA,
        pltpu.SemaphoreType.DMA,
        pltpu.SemaphoreType.DMA,
        pltpu.SemaphoreType.DMA,
        pltpu.SemaphoreType.DMA,
    ],
)(_sc_body)


def kernel(user_emb, item_emb, edge_index, edge_weight):
    er3 = edge_index.reshape(2, E_ROWS, CHUNK)
    user_o, item_o, _, _ = _sc_kernel(user_emb, item_emb, er3, edge_weight)
    return user_o, item_o
